# initial kernel scaffold (unmeasured)
import jax
import jax.numpy as jnp
from jax import lax
from jax.experimental import pallas as pl
from jax.experimental.pallas import tpu as pltpu

N_DEV = 4


def kernel(x, w_mat, scale_x, scale_w):
    m_per, k = x.shape
    _, n_per = w_mat.shape

    def body(x_ref, w_ref, sx_ref, sw_ref, out_ref, gbuf, send_sems, recv_sems):
        my = lax.axis_index("i")
        left = lax.rem(my + (N_DEV - 1), N_DEV)
        right = lax.rem(my + 1, N_DEV)

        barrier_sem = pltpu.get_barrier_semaphore()
        for nbr in (left, right):
            pl.semaphore_signal(
                barrier_sem, inc=1,
                device_id=(nbr,), device_id_type=pl.DeviceIdType.MESH,
            )
        pl.semaphore_wait(barrier_sem, 2)

        s = sx_ref[0] * sw_ref[0]

        def gemm_store(chunk, origin):
            acc = lax.dot_general(
                chunk, w_ref[...],
                (((1,), (0,)), ((), ())),
                preferred_element_type=jnp.int32,
            )
            y = jnp.maximum(acc.astype(jnp.float32) * s, 0.0)
            out_ref[pl.ds(origin * m_per, m_per), :] = y

        gemm_store(x_ref[...], my)

        for h in range(N_DEV - 1):
            src = x_ref if h == 0 else gbuf.at[h - 1]
            rdma = pltpu.make_async_remote_copy(
                src_ref=src,
                dst_ref=gbuf.at[h],
                send_sem=send_sems.at[h],
                recv_sem=recv_sems.at[h],
                device_id=(right,),
                device_id_type=pl.DeviceIdType.MESH,
            )
            rdma.start()
            rdma.wait()
            origin = lax.rem(my + (N_DEV - 1 - h), N_DEV)
            gemm_store(gbuf[h], origin)

    return pl.pallas_call(
        body,
        out_shape=jax.ShapeDtypeStruct((N_DEV * m_per, n_per), jnp.float32),
        in_specs=[
            pl.BlockSpec(memory_space=pltpu.VMEM),
            pl.BlockSpec(memory_space=pltpu.VMEM),
            pl.BlockSpec(memory_space=pltpu.SMEM),
            pl.BlockSpec(memory_space=pltpu.SMEM),
        ],
        out_specs=pl.BlockSpec(memory_space=pltpu.VMEM),
        scratch_shapes=[
            pltpu.VMEM((N_DEV - 1, m_per, k), jnp.int8),
            pltpu.SemaphoreType.DMA((N_DEV - 1,)),
            pltpu.SemaphoreType.DMA((N_DEV - 1,)),
        ],
        compiler_params=pltpu.CompilerParams(collective_id=0),
    )(x, w_mat, scale_x, scale_w)


# baseline (device time: 252127 ns/iter reference)
import jax
import jax.numpy as jnp
from jax import lax
from jax.experimental import pallas as pl
from jax.experimental.pallas import tpu as pltpu

N_DEV = 4
TM = 256


def kernel(x, w_mat, scale_x, scale_w):
    m_per, k = x.shape
    _, n_per = w_mat.shape
    n_tiles = m_per // TM

    def body(x_ref, w_ref, sx_ref, sw_ref, out_ref,
             gbuf, ybuf, send_sems, recv_sems, out_sems):
        my = lax.axis_index("i")
        left = lax.rem(my + (N_DEV - 1), N_DEV)
        right = lax.rem(my + 1, N_DEV)

        barrier_sem = pltpu.get_barrier_semaphore()
        for nbr in (left, right):
            pl.semaphore_signal(
                barrier_sem, inc=1,
                device_id=(nbr,), device_id_type=pl.DeviceIdType.MESH,
            )
        pl.semaphore_wait(barrier_sem, 2)

        s = sx_ref[0] * sw_ref[0]

        pending = [None, None]
        tile_ctr = [0]

        def gemm_chunk(chunk_ref, origin):
            for t in range(n_tiles):
                tile = chunk_ref[pl.ds(t * TM, TM), :]
                acc = lax.dot_general(
                    tile, w_ref[...],
                    (((1,), (0,)), ((), ())),
                    preferred_element_type=jnp.int32,
                )
                y = jnp.maximum(acc.astype(jnp.float32) * s, 0.0)
                slot = tile_ctr[0] % 2
                if pending[slot] is not None:
                    pending[slot].wait()
                ybuf[slot] = y
                cp = pltpu.make_async_copy(
                    ybuf.at[slot],
                    out_ref.at[pl.ds(origin * m_per + t * TM, TM), :],
                    out_sems.at[slot],
                )
                cp.start()
                pending[slot] = cp
                tile_ctr[0] += 1

        gemm_chunk(x_ref, my)

        for h in range(N_DEV - 1):
            src = x_ref if h == 0 else gbuf.at[h - 1]
            rdma = pltpu.make_async_remote_copy(
                src_ref=src,
                dst_ref=gbuf.at[h],
                send_sem=send_sems.at[h],
                recv_sem=recv_sems.at[h],
                device_id=(right,),
                device_id_type=pl.DeviceIdType.MESH,
            )
            rdma.start()
            rdma.wait()
            origin = lax.rem(my + (N_DEV - 1 - h), N_DEV)
            gemm_chunk(gbuf.at[h], origin)

        for p in pending:
            if p is not None:
                p.wait()

    return pl.pallas_call(
        body,
        out_shape=jax.ShapeDtypeStruct((N_DEV * m_per, n_per), jnp.float32),
        in_specs=[
            pl.BlockSpec(memory_space=pltpu.VMEM),
            pl.BlockSpec(memory_space=pltpu.VMEM),
            pl.BlockSpec(memory_space=pltpu.SMEM),
            pl.BlockSpec(memory_space=pltpu.SMEM),
        ],
        out_specs=pl.BlockSpec(memory_space=pl.ANY),
        scratch_shapes=[
            pltpu.VMEM((N_DEV - 1, m_per, k), jnp.int8),
            pltpu.VMEM((2, TM, n_per), jnp.float32),
            pltpu.SemaphoreType.DMA((N_DEV - 1,)),
            pltpu.SemaphoreType.DMA((N_DEV - 1,)),
            pltpu.SemaphoreType.DMA((2,)),
        ],
        compiler_params=pltpu.CompilerParams(collective_id=0),
    )(x, w_mat, scale_x, scale_w)


# device time: 121749 ns/iter; 2.0709x vs baseline; 2.0709x over previous
import jax
import jax.numpy as jnp
from jax import lax
from jax.experimental import pallas as pl
from jax.experimental.pallas import tpu as pltpu

N_DEV = 4
TM = 256


def kernel(x, w_mat, scale_x, scale_w):
    m_per, k = x.shape
    _, n_per = w_mat.shape
    half = m_per // 2

    def body(x_ref, w_ref, sx_ref, sw_ref, out_ref,
             lbuf, rbuf, obuf, ybuf, send_sems, recv_sems, out_sems):
        my = lax.axis_index("i")
        left = lax.rem(my + (N_DEV - 1), N_DEV)
        right = lax.rem(my + 1, N_DEV)
        opp = lax.rem(my + 2, N_DEV)

        barrier_sem = pltpu.get_barrier_semaphore()
        for nbr in (left, right):
            pl.semaphore_signal(
                barrier_sem, inc=1,
                device_id=(nbr,), device_id_type=pl.DeviceIdType.MESH,
            )
        pl.semaphore_wait(barrier_sem, 2)

        s = sx_ref[0] * sw_ref[0]

        def rdma(src, dst, sem, dev):
            return pltpu.make_async_remote_copy(
                src_ref=src, dst_ref=dst,
                send_sem=send_sems.at[sem], recv_sem=recv_sems.at[sem],
                device_id=(dev,), device_id_type=pl.DeviceIdType.MESH,
            )

        p1 = [
            rdma(x_ref.at[0:half], lbuf.at[0], 0, right),
            rdma(x_ref.at[half:m_per], lbuf.at[1], 1, right),
            rdma(x_ref.at[0:half], rbuf.at[0], 2, left),
            rdma(x_ref.at[half:m_per], rbuf.at[1], 3, left),
        ]
        for r in p1:
            r.start()

        pending = [None, None]
        tile_ctr = [0]

        def gemm_rows(chunk_ref, nrows, out_row0):
            for t in range(nrows // TM):
                tile = chunk_ref[pl.ds(t * TM, TM), :]
                acc = lax.dot_general(
                    tile, w_ref[...],
                    (((1,), (0,)), ((), ())),
                    preferred_element_type=jnp.int32,
                )
                y = jnp.maximum(acc.astype(jnp.float32) * s, 0.0)
                slot = tile_ctr[0] % 2
                if pending[slot] is not None:
                    pending[slot].wait()
                ybuf[slot] = y
                cp = pltpu.make_async_copy(
                    ybuf.at[slot],
                    out_ref.at[pl.ds(out_row0 + t * TM, TM), :],
                    out_sems.at[slot],
                )
                cp.start()
                pending[slot] = cp
                tile_ctr[0] += 1

        gemm_rows(x_ref, m_per, my * m_per)

        p1[0].wait_recv()
        s4 = rdma(lbuf.at[0], obuf.at[0], 4, right)
        s4.start()
        gemm_rows(lbuf.at[0], half, left * m_per)

        p1[2].wait_recv()
        gemm_rows(rbuf.at[0], half, right * m_per)

        p1[3].wait_recv()
        s5 = rdma(rbuf.at[1], obuf.at[1], 5, left)
        s5.start()

        p1[1].wait_recv()
        gemm_rows(lbuf.at[1], half, left * m_per + half)
        gemm_rows(rbuf.at[1], half, right * m_per + half)

        s4.wait_recv()
        gemm_rows(obuf.at[0], half, opp * m_per)
        s5.wait_recv()
        gemm_rows(obuf.at[1], half, opp * m_per + half)

        for r in p1:
            r.wait_send()
        s4.wait_send()
        s5.wait_send()
        for p in pending:
            if p is not None:
                p.wait()

    return pl.pallas_call(
        body,
        out_shape=jax.ShapeDtypeStruct((N_DEV * m_per, n_per), jnp.float32),
        in_specs=[
            pl.BlockSpec(memory_space=pltpu.VMEM),
            pl.BlockSpec(memory_space=pltpu.VMEM),
            pl.BlockSpec(memory_space=pltpu.SMEM),
            pl.BlockSpec(memory_space=pltpu.SMEM),
        ],
        out_specs=pl.BlockSpec(memory_space=pl.ANY),
        scratch_shapes=[
            pltpu.VMEM((2, half, k), jnp.int8),
            pltpu.VMEM((2, half, k), jnp.int8),
            pltpu.VMEM((2, half, k), jnp.int8),
            pltpu.VMEM((2, TM, n_per), jnp.float32),
            pltpu.SemaphoreType.DMA((6,)),
            pltpu.SemaphoreType.DMA((6,)),
            pltpu.SemaphoreType.DMA((2,)),
        ],
        compiler_params=pltpu.CompilerParams(collective_id=0),
    )(x, w_mat, scale_x, scale_w)


# device time: 118205 ns/iter; 2.1330x vs baseline; 1.0300x over previous
import jax
import jax.numpy as jnp
from jax import lax
from jax.experimental import pallas as pl
from jax.experimental.pallas import tpu as pltpu

N_DEV = 4
TM = 256


def kernel(x, w_mat, scale_x, scale_w):
    m_per, k = x.shape
    _, n_per = w_mat.shape
    half = m_per // 2

    def body(x_ref, w_ref, sx_ref, sw_ref, out_ref,
             lbuf, rbuf, obuf, ybuf, send_sems, recv_sems, out_sems):
        my = lax.axis_index("i")
        left = lax.rem(my + (N_DEV - 1), N_DEV)
        right = lax.rem(my + 1, N_DEV)
        opp = lax.rem(my + 2, N_DEV)

        barrier_sem = pltpu.get_barrier_semaphore()
        for nbr in (left, right):
            pl.semaphore_signal(
                barrier_sem, inc=1,
                device_id=(nbr,), device_id_type=pl.DeviceIdType.MESH,
            )
        pl.semaphore_wait(barrier_sem, 2)

        s = sx_ref[0] * sw_ref[0]

        def rdma(src, dst, sem, dev):
            return pltpu.make_async_remote_copy(
                src_ref=src, dst_ref=dst,
                send_sem=send_sems.at[sem], recv_sem=recv_sems.at[sem],
                device_id=(dev,), device_id_type=pl.DeviceIdType.MESH,
            )

        p1 = [
            rdma(x_ref.at[0:half], lbuf.at[0], 0, right),
            rdma(x_ref.at[half:m_per], lbuf.at[1], 1, right),
            rdma(x_ref.at[0:half], rbuf.at[0], 2, left),
            rdma(x_ref.at[half:m_per], rbuf.at[1], 3, left),
        ]
        for r in p1:
            r.start()

        pending = [None, None]
        tile_ctr = [0]

        def gemm_rows(chunk_ref, nrows, out_row0):
            for t in range(nrows // TM):
                tile = chunk_ref[pl.ds(t * TM, TM), :]
                acc = lax.dot_general(
                    tile, w_ref[...],
                    (((1,), (0,)), ((), ())),
                    preferred_element_type=jnp.int32,
                )
                y = jnp.maximum(acc.astype(jnp.float32) * s, 0.0)
                slot = tile_ctr[0] % 2
                if pending[slot] is not None:
                    pending[slot].wait()
                ybuf[slot] = y
                cp = pltpu.make_async_copy(
                    ybuf.at[slot],
                    out_ref.at[pl.ds(out_row0 + t * TM, TM), :],
                    out_sems.at[slot],
                )
                cp.start()
                pending[slot] = cp
                tile_ctr[0] += 1

        gemm_rows(x_ref, m_per, my * m_per)

        q = half // 2
        p1[0].wait_recv()
        s4 = [rdma(lbuf.at[0, 0:q], obuf.at[0], 4, right),
              rdma(lbuf.at[0, q:half], obuf.at[1], 5, right)]
        for r in s4:
            r.start()
        gemm_rows(lbuf.at[0], half, left * m_per)

        p1[2].wait_recv()
        gemm_rows(rbuf.at[0], half, right * m_per)

        p1[3].wait_recv()
        s5 = [rdma(rbuf.at[1, 0:q], obuf.at[2], 6, left),
              rdma(rbuf.at[1, q:half], obuf.at[3], 7, left)]
        for r in s5:
            r.start()

        p1[1].wait_recv()
        gemm_rows(lbuf.at[1], half, left * m_per + half)
        gemm_rows(rbuf.at[1], half, right * m_per + half)

        s4[0].wait_recv()
        gemm_rows(obuf.at[0], q, opp * m_per)
        s5[0].wait_recv()
        gemm_rows(obuf.at[2], q, opp * m_per + half)
        s4[1].wait_recv()
        gemm_rows(obuf.at[1], q, opp * m_per + q)
        s5[1].wait_recv()
        gemm_rows(obuf.at[3], q, opp * m_per + half + q)

        for r in p1 + s4 + s5:
            r.wait_send()
        for p in pending:
            if p is not None:
                p.wait()

    return pl.pallas_call(
        body,
        out_shape=jax.ShapeDtypeStruct((N_DEV * m_per, n_per), jnp.float32),
        in_specs=[
            pl.BlockSpec(memory_space=pltpu.VMEM),
            pl.BlockSpec(memory_space=pltpu.VMEM),
            pl.BlockSpec(memory_space=pltpu.SMEM),
            pl.BlockSpec(memory_space=pltpu.SMEM),
        ],
        out_specs=pl.BlockSpec(memory_space=pl.ANY),
        scratch_shapes=[
            pltpu.VMEM((2, half, k), jnp.int8),
            pltpu.VMEM((2, half, k), jnp.int8),
            pltpu.VMEM((4, half // 2, k), jnp.int8),
            pltpu.VMEM((2, TM, n_per), jnp.float32),
            pltpu.SemaphoreType.DMA((8,)),
            pltpu.SemaphoreType.DMA((8,)),
            pltpu.SemaphoreType.DMA((2,)),
        ],
        compiler_params=pltpu.CompilerParams(collective_id=0),
    )(x, w_mat, scale_x, scale_w)
